# baseline (speedup 1.0000x reference)
import functools

import jax
import jax.numpy as jnp
from jax import lax
from jax.experimental import pallas as pl
from jax.experimental.pallas import tpu as pltpu
from jax.experimental.pallas import tpu_sc as plsc

N_ITEMS = 1_000_000
D = 16
BATCH = 16384
HIST = 50

NC = 2
NS = 16
NW = NC * NS

TCH = 1792
N_TCH = N_ITEMS // TCH
TAIL0 = N_TCH * TCH
N_SLOTS = 17

USERS_PW = BATCH // NW
BLK_U = 64
BLK_R = BLK_U * HIST
N_BLK = USERS_PW // BLK_U
GROUPS = BLK_R // D
CHUNK = 128
N_CHUNK = BLK_R // CHUNK


def _transpose_body(itT_hbm, tail_hbm, out_hbm, in_v, out_v,
                    isem0, isem1, osem0, osem1):
    wid = lax.axis_index("s") * NC + lax.axis_index("c")
    iota16 = lax.iota(jnp.int32, 16)
    isems = (isem0, isem1)
    osems = (osem0, osem1)

    iota16x16 = iota16 * D

    def transpose_groups(inb, outb, width):
        def group(cb, carry):
            base = cb * (16 * D)
            for d in range(D):
                v = inb[d, pl.ds(cb * 16, 16)]
                plsc.store_scatter(outb, [base + iota16x16 + d], v)
            return carry

        lax.fori_loop(0, width // 16, group, 0, unroll=False)

    inflight = {}
    outflight = {}

    def prefetch(j, p):
        c0 = (j * NW + wid) * TCH
        inflight[p] = [
            pltpu.async_copy(itT_hbm.at[pl.ds(0, 8), pl.ds(c0, TCH)],
                             in_v.at[pl.ds(p * D, 8)], isems[p]),
            pltpu.async_copy(itT_hbm.at[pl.ds(8, 8), pl.ds(c0, TCH)],
                             in_v.at[pl.ds(p * D + 8, 8)], isems[p]),
        ]

    prefetch(0, 0)
    for j in range(N_SLOTS):
        p = j % 2
        if j + 1 < N_SLOTS:
            prefetch(j + 1, 1 - p)
        for c in inflight.pop(p):
            c.wait()
        if p in outflight:
            outflight.pop(p).wait()
        transpose_groups(in_v.at[pl.ds(p * D, D)],
                         out_v.at[pl.ds(p * TCH * D, TCH * D)], TCH)
        c0 = (j * NW + wid) * TCH
        outflight[p] = pltpu.async_copy(
            out_v.at[pl.ds(p * TCH * D, TCH * D)],
            out_hbm.at[pl.ds(c0 * D, TCH * D)], osems[p])

    for c in outflight.values():
        c.wait()

    def do_chunk_sync(c0, width):
        copies = [
            pltpu.async_copy(itT_hbm.at[pl.ds(0, 8), pl.ds(c0, width)],
                             in_v.at[pl.ds(0, 8), pl.ds(0, width)],
                             isems[0]),
            pltpu.async_copy(itT_hbm.at[pl.ds(8, 8), pl.ds(c0, width)],
                             in_v.at[pl.ds(8, 8), pl.ds(0, width)],
                             isems[0]),
        ]
        for c in copies:
            c.wait()
        transpose_groups(in_v.at[pl.ds(0, D)],
                         out_v.at[pl.ds(0, width * D)], width)
        pltpu.sync_copy(out_v.at[pl.ds(0, width * D)],
                        out_hbm.at[pl.ds(c0 * D, width * D)])

    @pl.when(wid < N_TCH - N_SLOTS * NW)
    def _extra():
        do_chunk_sync((N_SLOTS * NW + wid) * TCH, TCH)

    @pl.when(wid == NW - 1)
    def _t2():
        tail_v = out_v.at[pl.ds(0, 64 * D)]
        pltpu.sync_copy(tail_hbm, tail_v)
        pltpu.sync_copy(tail_v, out_hbm.at[pl.ds(TAIL0 * D, 64 * D)])


def _gather_body(u_hbm, items_hbm, itab_hbm, out_hbm,
                 idx_v, u_v, it_v, out_v, gsem0, gsem1, osem0, osem1):
    wid = lax.axis_index("s") * NC + lax.axis_index("c")
    iota16 = lax.iota(jnp.int32, 16)
    row_base0 = wid * (USERS_PW * HIST)
    u_base0 = wid * USERS_PW
    gsems = (gsem0, gsem1)
    osems = (osem0, osem1)

    gather_copies = {}

    def prefetch(b, p):
        row_base = row_base0 + b * BLK_R
        pltpu.sync_copy(items_hbm.at[pl.ds(row_base, BLK_R)], idx_v.at[p])
        copies = [pltpu.async_copy(u_hbm.at[pl.ds(u_base0 + b * BLK_U, BLK_U)],
                                   u_v.at[p], gsems[p])]
        for j in range(N_CHUNK):
            copies.append(pltpu.async_copy(
                itab_hbm.at[idx_v.at[p].at[pl.ds(j * CHUNK, CHUNK)]],
                it_v.at[p].at[pl.ds(j * CHUNK, CHUNK)],
                gsems[p]))
        gather_copies[p] = copies

    store_copies = {}

    prefetch(0, 0)
    for b in range(N_BLK):
        p = b % 2
        if b + 1 < N_BLK:
            prefetch(b + 1, 1 - p)
        for c in gather_copies.pop(p):
            c.wait()
        if p in store_copies:
            store_copies.pop(p).wait()

        itb = it_v.at[p]
        ub = u_v.at[p]
        outb = out_v.at[p]

        def group(g, carry, itb=itb, ub=ub, outb=outb):
            rows = g * 16 + iota16
            urows = rows // HIST
            accs = [jnp.zeros((16,), jnp.float32) for _ in range(4)]
            for d in range(D):
                dcol = jnp.full((16,), d, jnp.int32)
                ic = plsc.load_gather(itb, [rows, dcol])
                uc = plsc.load_gather(ub, [urows, dcol])
                accs[d % 4] = accs[d % 4] + ic * uc
            outb[pl.ds(g * 16, 16)] = (accs[0] + accs[1]) + (accs[2] + accs[3])
            return carry

        lax.fori_loop(0, GROUPS, group, 0, unroll=2)
        store_copies[p] = pltpu.async_copy(
            outb, out_hbm.at[pl.ds(row_base0 + b * BLK_R, BLK_R)], osems[p])

    for c in store_copies.values():
        c.wait()


@jax.jit
def _mf(users, items_flat, user_table, item_table):
    mesh = plsc.VectorSubcoreMesh(core_axis_name="c", subcore_axis_name="s",
                                  num_cores=NC, num_subcores=NS)
    cparams = pltpu.CompilerParams(
        needs_layout_passes=False, use_tc_tiling_on_sc=False)

    tail = lax.slice(item_table, (TAIL0, 0), (N_ITEMS, D))
    itab_flat = pl.kernel(
        _transpose_body,
        out_type=jax.ShapeDtypeStruct((N_ITEMS * D,), jnp.float32),
        mesh=mesh,
        scratch_types=[
            pltpu.VMEM((2 * D, TCH), jnp.float32),
            pltpu.VMEM((2 * TCH * D,), jnp.float32),
            pltpu.SemaphoreType.DMA,
            pltpu.SemaphoreType.DMA,
            pltpu.SemaphoreType.DMA,
            pltpu.SemaphoreType.DMA,
        ],
        compiler_params=pltpu.CompilerParams(
            needs_layout_passes=False, use_tc_tiling_on_sc=True),
    )(item_table.T, tail.reshape(-1))
    itab_rows = itab_flat.reshape(N_ITEMS, D)

    u = jnp.take(user_table, users, axis=0)

    out_flat = pl.kernel(
        _gather_body,
        out_type=jax.ShapeDtypeStruct((BATCH * HIST,), jnp.float32),
        mesh=mesh,
        scratch_types=[
            pltpu.VMEM((2, BLK_R), jnp.int32),
            pltpu.VMEM((2, BLK_U, D), jnp.float32),
            pltpu.VMEM((2, BLK_R, D), jnp.float32),
            pltpu.VMEM((2, BLK_R), jnp.float32),
            pltpu.SemaphoreType.DMA,
            pltpu.SemaphoreType.DMA,
            pltpu.SemaphoreType.DMA,
            pltpu.SemaphoreType.DMA,
        ],
        compiler_params=cparams,
    )(u, items_flat, itab_rows)

    return out_flat


def kernel(users, items, user_table, item_table):
    users = users.astype(jnp.int32)
    items_flat = items.astype(jnp.int32).reshape(-1)
    out_flat = _mf(users, items_flat, user_table, item_table)
    return out_flat.reshape(BATCH, HIST)


# speedup vs baseline: 1.0337x; 1.0337x over previous
import jax
import jax.numpy as jnp
from jax import lax
from jax.experimental import pallas as pl
from jax.experimental.pallas import tpu as pltpu
from jax.experimental.pallas import tpu_sc as plsc

N_ITEMS = 1_000_000
D = 16
BATCH = 16384
HIST = 50

NC = 2
NS = 16
NW = NC * NS

TCH = 1792
N_TCH = N_ITEMS // TCH
TAIL0 = N_TCH * TCH
N_SLOTS = 17

USERS_PW = BATCH // NW
BLK_U = 64
BLK_R = BLK_U * HIST
N_BLK = USERS_PW // BLK_U
GROUPS = BLK_R // D
CHUNK = 128
N_CHUNK = BLK_R // CHUNK


def _transpose_body(itT_hbm, tail_hbm, out_hbm, in_v, out_v,
                    isem0, isem1, osem0, osem1):
    wid = lax.axis_index("s") * NC + lax.axis_index("c")
    iota16 = lax.iota(jnp.int32, 16)
    isems = (isem0, isem1)
    osems = (osem0, osem1)

    iota16x16 = iota16 * D

    def transpose_groups(inb, outb, width):
        def group(cb, carry):
            base = cb * (16 * D)
            for d in range(D):
                v = inb[d, pl.ds(cb * 16, 16)]
                plsc.store_scatter(outb, [base + iota16x16 + d], v)
            return carry

        lax.fori_loop(0, width // 16, group, 0, unroll=False)

    inflight = {}
    outflight = {}

    def prefetch(j, p):
        c0 = (j * NW + wid) * TCH
        inflight[p] = [
            pltpu.async_copy(itT_hbm.at[pl.ds(0, 8), pl.ds(c0, TCH)],
                             in_v.at[pl.ds(p * D, 8)], isems[p]),
            pltpu.async_copy(itT_hbm.at[pl.ds(8, 8), pl.ds(c0, TCH)],
                             in_v.at[pl.ds(p * D + 8, 8)], isems[p]),
        ]

    prefetch(0, 0)
    for j in range(N_SLOTS):
        p = j % 2
        if j + 1 < N_SLOTS:
            prefetch(j + 1, 1 - p)
        for c in inflight.pop(p):
            c.wait()
        if p in outflight:
            outflight.pop(p).wait()
        transpose_groups(in_v.at[pl.ds(p * D, D)],
                         out_v.at[pl.ds(p * TCH * D, TCH * D)], TCH)
        c0 = (j * NW + wid) * TCH
        outflight[p] = pltpu.async_copy(
            out_v.at[pl.ds(p * TCH * D, TCH * D)],
            out_hbm.at[pl.ds(c0 * D, TCH * D)], osems[p])

    for c in outflight.values():
        c.wait()

    def do_chunk_sync(c0, width):
        copies = [
            pltpu.async_copy(itT_hbm.at[pl.ds(0, 8), pl.ds(c0, width)],
                             in_v.at[pl.ds(0, 8), pl.ds(0, width)],
                             isems[0]),
            pltpu.async_copy(itT_hbm.at[pl.ds(8, 8), pl.ds(c0, width)],
                             in_v.at[pl.ds(8, 8), pl.ds(0, width)],
                             isems[0]),
        ]
        for c in copies:
            c.wait()
        transpose_groups(in_v.at[pl.ds(0, D)],
                         out_v.at[pl.ds(0, width * D)], width)
        pltpu.sync_copy(out_v.at[pl.ds(0, width * D)],
                        out_hbm.at[pl.ds(c0 * D, width * D)])

    @pl.when(wid < N_TCH - N_SLOTS * NW)
    def _extra():
        do_chunk_sync((N_SLOTS * NW + wid) * TCH, TCH)

    @pl.when(wid == NW - 1)
    def _t2():
        tail_v = out_v.at[pl.ds(0, 64 * D)]
        pltpu.sync_copy(tail_hbm, tail_v)
        pltpu.sync_copy(tail_v, out_hbm.at[pl.ds(TAIL0 * D, 64 * D)])


def _gather_body(u_hbm, items_hbm, itab_hbm, out_hbm,
                 idx_v, u_v, it_v, out_v, gsem0, gsem1, osem0, osem1):
    wid = lax.axis_index("s") * NC + lax.axis_index("c")
    iota16 = lax.iota(jnp.int32, 16)
    row_base0 = wid * (USERS_PW * HIST)
    u_base0 = wid * USERS_PW
    gsems = (gsem0, gsem1)
    osems = (osem0, osem1)

    gather_copies = {}

    def prefetch(b, p):
        row_base = row_base0 + b * BLK_R
        pltpu.sync_copy(items_hbm.at[pl.ds(row_base, BLK_R)], idx_v.at[p])
        copies = [pltpu.async_copy(u_hbm.at[pl.ds(u_base0 + b * BLK_U, BLK_U)],
                                   u_v.at[p], gsems[p])]
        for j in range(N_CHUNK):
            copies.append(pltpu.async_copy(
                itab_hbm.at[idx_v.at[p].at[pl.ds(j * CHUNK, CHUNK)]],
                it_v.at[p].at[pl.ds(j * CHUNK, CHUNK)],
                gsems[p]))
        gather_copies[p] = copies

    store_copies = {}

    prefetch(0, 0)
    for b in range(N_BLK):
        p = b % 2
        if b + 1 < N_BLK:
            prefetch(b + 1, 1 - p)
        for c in gather_copies.pop(p):
            c.wait()
        if p in store_copies:
            store_copies.pop(p).wait()

        itb = it_v.at[p]
        ub = u_v.at[p]
        outb = out_v.at[p]

        def group(g, carry, itb=itb, ub=ub, outb=outb):
            rows = g * 16 + iota16
            urows = rows // HIST
            acc0 = jnp.zeros((16,), jnp.float32)
            acc1 = jnp.zeros((16,), jnp.float32)
            for d in range(D):
                dcol = jnp.full((16,), d, jnp.int32)
                ic = plsc.load_gather(itb, [rows, dcol])
                uc = plsc.load_gather(ub, [urows, dcol])
                if d % 2 == 0:
                    acc0 = acc0 + ic * uc
                else:
                    acc1 = acc1 + ic * uc
            outb[pl.ds(g * 16, 16)] = acc0 + acc1
            return carry

        lax.fori_loop(0, GROUPS, group, 0, unroll=False)
        store_copies[p] = pltpu.async_copy(
            outb, out_hbm.at[pl.ds(row_base0 + b * BLK_R, BLK_R)], osems[p])

    for c in store_copies.values():
        c.wait()


@jax.jit
def _mf(users, items_flat, user_table, item_table):
    mesh = plsc.VectorSubcoreMesh(core_axis_name="c", subcore_axis_name="s",
                                  num_cores=NC, num_subcores=NS)
    cparams = pltpu.CompilerParams(
        needs_layout_passes=False, use_tc_tiling_on_sc=False)

    tail = lax.slice(item_table, (TAIL0, 0), (N_ITEMS, D))
    itab_flat = pl.kernel(
        _transpose_body,
        out_type=jax.ShapeDtypeStruct((N_ITEMS * D,), jnp.float32),
        mesh=mesh,
        scratch_types=[
            pltpu.VMEM((2 * D, TCH), jnp.float32),
            pltpu.VMEM((2 * TCH * D,), jnp.float32),
            pltpu.SemaphoreType.DMA,
            pltpu.SemaphoreType.DMA,
            pltpu.SemaphoreType.DMA,
            pltpu.SemaphoreType.DMA,
        ],
        compiler_params=pltpu.CompilerParams(
            needs_layout_passes=False, use_tc_tiling_on_sc=True),
    )(item_table.T, tail.reshape(-1))
    itab_rows = itab_flat.reshape(N_ITEMS, D)

    u = jnp.take(user_table, users, axis=0)

    out_flat = pl.kernel(
        _gather_body,
        out_type=jax.ShapeDtypeStruct((BATCH * HIST,), jnp.float32),
        mesh=mesh,
        scratch_types=[
            pltpu.VMEM((2, BLK_R), jnp.int32),
            pltpu.VMEM((2, BLK_U, D), jnp.float32),
            pltpu.VMEM((2, BLK_R, D), jnp.float32),
            pltpu.VMEM((2, BLK_R), jnp.float32),
            pltpu.SemaphoreType.DMA,
            pltpu.SemaphoreType.DMA,
            pltpu.SemaphoreType.DMA,
            pltpu.SemaphoreType.DMA,
        ],
        compiler_params=cparams,
    )(u, items_flat, itab_rows)

    return out_flat


def kernel(users, items, user_table, item_table):
    users = users.astype(jnp.int32)
    items_flat = items.astype(jnp.int32).reshape(-1)
    out_flat = _mf(users, items_flat, user_table, item_table)
    return out_flat.reshape(BATCH, HIST)
